# baseline (speedup 1.0000x reference)
import jax
import jax.numpy as jnp
from jax.experimental import pallas as pl
from jax.experimental.pallas import tpu as pltpu

E = 8
TOPK = 2
EPS = 1e-5

_INTERPRET = False


def _route_body(pooled_ref, wr_ref, topi_ref, gates_ref):
    logits = jax.lax.dot_general(
        pooled_ref[...], wr_ref[...], (((1,), (1,)), ((), ())),
        preferred_element_type=jnp.float32)
    B = logits.shape[0]
    iota = jax.lax.broadcasted_iota(jnp.int32, (B, E), 1)
    m1 = jnp.max(logits, axis=1, keepdims=True)
    i1 = jnp.min(jnp.where(logits == m1, iota, E), axis=1, keepdims=True)
    masked = jnp.where(iota == i1, -jnp.inf, logits)
    m2 = jnp.max(masked, axis=1, keepdims=True)
    i2 = jnp.min(jnp.where(masked == m2, iota, E), axis=1, keepdims=True)
    g0 = jax.nn.sigmoid(m1 - m2)
    topi_ref[...] = jnp.concatenate([i1, i2], axis=1)
    gates_ref[...] = jnp.concatenate([g0, 1.0 - g0], axis=1)


def _moe_body(topi_s, gates_s, x_ref, wsh_ref, bsh_ref, w1_ref, b1_ref,
              w2_ref, out_ref):
    b = pl.program_id(0)
    xb = x_ref[...]

    def expert(k):
        e = topi_s[b, k]
        h = jnp.dot(w1_ref[e], xb, preferred_element_type=jnp.float32)
        bias = b1_ref[pl.ds(e, 1), :].reshape(h.shape[0], 1)
        h = h + bias
        h = h * jax.nn.sigmoid(h)
        return jnp.dot(w2_ref[e], h, preferred_element_type=jnp.float32)

    o1 = expert(0)
    o2 = expert(1)
    hs = jnp.dot(wsh_ref[...], xb, preferred_element_type=jnp.float32)
    hs = hs + bsh_ref[...]
    out_ref[...] = (hs * jax.nn.sigmoid(hs)
                    + gates_s[b, 0] * o1 + gates_s[b, 1] * o2)


def kernel(x, Wr, Wsh, gsh, bsh, W1, g1, b1, W2):
    B, C, H, W = x.shape
    HW = H * W
    COUT = Wsh.shape[0]
    HID = W1.shape[1]

    pooled = jnp.mean(x, axis=(2, 3))
    x2 = x.reshape(B * C, HW)

    topi, gates = pl.pallas_call(
        _route_body,
        in_specs=[
            pl.BlockSpec((B, C), lambda: (0, 0)),
            pl.BlockSpec((E, C), lambda: (0, 0)),
        ],
        out_specs=[
            pl.BlockSpec((B, TOPK), lambda: (0, 0)),
            pl.BlockSpec((B, TOPK), lambda: (0, 0)),
        ],
        out_shape=[
            jax.ShapeDtypeStruct((B, TOPK), jnp.int32),
            jax.ShapeDtypeStruct((B, TOPK), jnp.float32),
        ],
        interpret=_INTERPRET,
    )(pooled, Wr)

    inv = 1.0 / jnp.sqrt(1.0 + EPS)
    Wshp = Wsh * (gsh * inv)[:, None]
    W1p = W1 * (g1 * inv)[:, :, None]
    bsh2 = bsh[:, None]

    grid_spec = pltpu.PrefetchScalarGridSpec(
        num_scalar_prefetch=2,
        grid=(B,),
        in_specs=[
            pl.BlockSpec((C, HW), lambda b, ti, gs: (b, 0)),
            pl.BlockSpec((COUT, C), lambda b, ti, gs: (0, 0)),
            pl.BlockSpec((COUT, 1), lambda b, ti, gs: (0, 0)),
            pl.BlockSpec((E, HID, C), lambda b, ti, gs: (0, 0, 0)),
            pl.BlockSpec((E, HID), lambda b, ti, gs: (0, 0)),
            pl.BlockSpec((E, COUT, HID), lambda b, ti, gs: (0, 0, 0)),
        ],
        out_specs=pl.BlockSpec((COUT, HW), lambda b, ti, gs: (b, 0)),
    )
    out = pl.pallas_call(
        _moe_body,
        grid_spec=grid_spec,
        out_shape=jax.ShapeDtypeStruct((B * COUT, HW), jnp.float32),
        interpret=_INTERPRET,
    )(topi, gates, x2, Wshp, bsh2, W1p, b1, W2)

    return out.reshape(B, COUT, H, W)


# speedup vs baseline: 1.6229x; 1.6229x over previous
import jax
import jax.numpy as jnp
from jax.experimental import pallas as pl
from jax.experimental.pallas import tpu as pltpu

E = 8
TOPK = 2
EPS = 1e-5

_INTERPRET = False


def _route_body(pooled_ref, wr_ref, topi_ref, gates_ref):
    logits = jax.lax.dot_general(
        pooled_ref[...], wr_ref[...], (((1,), (1,)), ((), ())),
        preferred_element_type=jnp.float32)
    B = logits.shape[0]
    iota = jax.lax.broadcasted_iota(jnp.int32, (B, E), 1)
    m1 = jnp.max(logits, axis=1, keepdims=True)
    i1 = jnp.min(jnp.where(logits == m1, iota, E), axis=1, keepdims=True)
    masked = jnp.where(iota == i1, -jnp.inf, logits)
    m2 = jnp.max(masked, axis=1, keepdims=True)
    i2 = jnp.min(jnp.where(masked == m2, iota, E), axis=1, keepdims=True)
    g0 = jax.nn.sigmoid(m1 - m2)
    topi_ref[...] = jnp.concatenate([i1, i2], axis=1)
    gates_ref[...] = jnp.concatenate([g0, 1.0 - g0], axis=1)


def _moe_body(topi_s, gates_s, x_ref, wsh_ref, bsh_ref, w1_ref, b1_ref,
              w2_ref, out_ref):
    b = pl.program_id(0)
    xb = x_ref[0]

    def expert(k):
        e = topi_s[b, k]
        h = jnp.dot(w1_ref[e], xb, preferred_element_type=jnp.float32)
        bias = b1_ref[pl.ds(e, 1), :].reshape(h.shape[0], 1)
        h = h + bias
        h = h * jax.nn.sigmoid(h)
        return jnp.dot(w2_ref[e], h, preferred_element_type=jnp.float32)

    o1 = expert(0)
    o2 = expert(1)
    hs = jnp.dot(wsh_ref[...], xb, preferred_element_type=jnp.float32)
    hs = hs + bsh_ref[...]
    out_ref[0] = (hs * jax.nn.sigmoid(hs)
                  + gates_s[b, 0] * o1 + gates_s[b, 1] * o2)


def kernel(x, Wr, Wsh, gsh, bsh, W1, g1, b1, W2):
    B, C, H, W = x.shape
    HW = H * W
    COUT = Wsh.shape[0]
    HID = W1.shape[1]

    pooled = jnp.mean(x, axis=(2, 3))
    x3 = x.reshape(B, C, HW)

    topi, gates = pl.pallas_call(
        _route_body,
        in_specs=[
            pl.BlockSpec((B, C), lambda: (0, 0)),
            pl.BlockSpec((E, C), lambda: (0, 0)),
        ],
        out_specs=[
            pl.BlockSpec((B, TOPK), lambda: (0, 0)),
            pl.BlockSpec((B, TOPK), lambda: (0, 0)),
        ],
        out_shape=[
            jax.ShapeDtypeStruct((B, TOPK), jnp.int32),
            jax.ShapeDtypeStruct((B, TOPK), jnp.float32),
        ],
        interpret=_INTERPRET,
    )(pooled, Wr)

    inv = 1.0 / jnp.sqrt(1.0 + EPS)
    Wshp = Wsh * (gsh * inv)[:, None]
    W1p = W1 * (g1 * inv)[:, :, None]
    bsh2 = bsh[:, None]

    grid_spec = pltpu.PrefetchScalarGridSpec(
        num_scalar_prefetch=2,
        grid=(B,),
        in_specs=[
            pl.BlockSpec((1, C, HW), lambda b, ti, gs: (b, 0, 0)),
            pl.BlockSpec((COUT, C), lambda b, ti, gs: (0, 0)),
            pl.BlockSpec((COUT, 1), lambda b, ti, gs: (0, 0)),
            pl.BlockSpec((E, HID, C), lambda b, ti, gs: (0, 0, 0)),
            pl.BlockSpec((E, HID), lambda b, ti, gs: (0, 0)),
            pl.BlockSpec((E, COUT, HID), lambda b, ti, gs: (0, 0, 0)),
        ],
        out_specs=pl.BlockSpec((1, COUT, HW), lambda b, ti, gs: (b, 0, 0)),
    )
    out = pl.pallas_call(
        _moe_body,
        grid_spec=grid_spec,
        out_shape=jax.ShapeDtypeStruct((B, COUT, HW), jnp.float32),
        interpret=_INTERPRET,
    )(topi, gates, x3, Wshp, bsh2, W1p, b1, W2)

    return out.reshape(B, COUT, H, W)
